# baseline (speedup 1.0000x reference)
import functools

import numpy as np
import jax
import jax.numpy as jnp
from jax import lax
from jax.experimental import pallas as pl
from jax.experimental.pallas import tpu as pltpu
from jax.experimental.pallas import tpu_sc as plsc

VOCAB = 1000000
D = 64
B = 4096
L = 200
MAX_LEN = 512

NC = 2
NS = 16
NW = NC * NS

BW = B // NW
NBUF = 4
PREF = 2


def _pos_encoding(max_len, d):
    pos = np.arange(max_len, dtype=np.float32)[:, None]
    div = np.exp(np.arange(0, d, 2, dtype=np.float32) * (-np.log(10000.0) / d))
    pe = np.zeros((max_len, d), dtype=np.float32)
    pe[:, 0::2] = np.sin(pos * div)
    pe[:, 1::2] = np.cos(pos * div)
    return pe


_PE = jnp.asarray(_pos_encoding(MAX_LEN, D)[:L])
_SCALE = float(np.sqrt(np.float32(D)))


@functools.partial(
    pl.kernel,
    out_type=jax.ShapeDtypeStruct((B, L, D), jnp.float32),
    mesh=plsc.VectorSubcoreMesh(
        core_axis_name="c", subcore_axis_name="s", num_cores=NC, num_subcores=NS
    ),
    scratch_types=[
        [pltpu.VMEM((1, BW), jnp.int32) for _ in range(NBUF)],
        [pltpu.VMEM((BW, D), jnp.float32) for _ in range(NBUF)],
        pltpu.VMEM((L, D), jnp.float32),
        [pltpu.SemaphoreType.DMA for _ in range(NBUF)],
        [pltpu.SemaphoreType.DMA for _ in range(NBUF)],
    ],
    compiler_params=pltpu.CompilerParams(use_tc_tiling_on_sc=False),
)
def _emb_kernel(seq_t_hbm, w_hbm, pe_hbm, out_hbm,
                idx_bufs, rows_bufs, pe_v, gsems, osems):
    wid = lax.axis_index("s") * NC + lax.axis_index("c")
    b0 = wid * BW
    pltpu.sync_copy(pe_hbm, pe_v)

    def fire_gather(g, p):
        pltpu.sync_copy(
            seq_t_hbm.at[pl.ds(g, 1), pl.ds(b0, BW)], idx_bufs[p]
        )
        pltpu.async_copy(
            w_hbm.at[idx_bufs[p].at[0]], rows_bufs[p], gsems[p]
        )

    def wait_gather(p):
        pltpu.make_async_copy(
            w_hbm.at[pl.ds(0, BW)], rows_bufs[p], gsems[p]
        ).wait()

    def wait_out(p):
        pltpu.make_async_copy(
            rows_bufs[p], out_hbm.at[pl.ds(0, BW), 0], osems[p]
        ).wait()

    for g0 in range(PREF):
        fire_gather(g0, g0)

    def outer(h, carry):
        for p in range(NBUF):
            g = h * NBUF + p
            p2 = (p + PREF) % NBUF

            @pl.when(g + PREF < L)
            def _():
                @pl.when(g >= NBUF - PREF)
                def _():
                    wait_out(p2)
                fire_gather(g + PREF, p2)

            wait_gather(p)

            pes = [pe_v[g, pl.ds(j * 16, 16)] for j in range(D // 16)]

            def body(b, c):
                for j in range(D // 16):
                    sl = pl.ds(j * 16, 16)
                    rows_bufs[p][b, sl] = rows_bufs[p][b, sl] * _SCALE + pes[j]
                return c

            lax.fori_loop(0, BW, body, 0, unroll=4)

            pltpu.async_copy(
                rows_bufs[p], out_hbm.at[pl.ds(b0, BW), g], osems[p]
            )
        return carry

    lax.fori_loop(0, L // NBUF, outer, 0)

    for p in range(NBUF):
        wait_out(p)


def kernel(seq, weight):
    return _emb_kernel(seq.T, weight, _PE)
